# baseline (speedup 1.0000x reference)
import functools

import jax
import jax.numpy as jnp
from jax import lax
from jax.experimental import pallas as pl
from jax.experimental.pallas import tpu as pltpu
from jax.experimental.pallas import tpu_sc as plsc

_N, _D, _H, _E, _K = 4096, 1024, 1024, 8, 2
_TILE = 256
_T = 40
_P = _T * _TILE
_GT = 512
_NW = 32
_BPW = _P // _NW
_GCH = 64
_TPW = _N // _NW
_CCH = 32


def _gate_body(x_ref, wg1_ref, bg1_ref, wg2_ref, bg2_ref,
               probs_ref, idx_ref, w_ref):
    h = jnp.dot(x_ref[...], wg1_ref[...], preferred_element_type=jnp.float32)
    h = jnp.maximum(h + bg1_ref[...], 0.0)
    s = jnp.dot(h, wg2_ref[...], preferred_element_type=jnp.float32)
    s = s + bg2_ref[...]
    s = s - jnp.max(s, axis=-1, keepdims=True)
    es = jnp.exp(s)
    probs = es / jnp.sum(es, axis=-1, keepdims=True)
    probs_ref[...] = probs
    ii = lax.broadcasted_iota(jnp.int32, probs.shape, 1)
    p0 = jnp.max(probs, axis=-1, keepdims=True)
    i0 = jnp.min(jnp.where(probs == p0, ii, _E), axis=-1, keepdims=True)
    masked = jnp.where(ii == i0, -1.0, probs)
    p1 = jnp.max(masked, axis=-1, keepdims=True)
    i1 = jnp.min(jnp.where(masked == p1, ii, _E), axis=-1, keepdims=True)
    tot = p0 + p1
    idx_ref[...] = jnp.concatenate([i0, i1], axis=1)
    w_ref[...] = jnp.concatenate([p0 / tot, p1 / tot], axis=1)


def _gate(x, Wg1, bg1, Wg2, bg2):
    return pl.pallas_call(
        _gate_body,
        grid=(_N // _GT,),
        in_specs=[
            pl.BlockSpec((_GT, _D), lambda i: (i, 0)),
            pl.BlockSpec((_D, _H), lambda i: (0, 0)),
            pl.BlockSpec((1, _H), lambda i: (0, 0)),
            pl.BlockSpec((_H, _E), lambda i: (0, 0)),
            pl.BlockSpec((1, _E), lambda i: (0, 0)),
        ],
        out_specs=[
            pl.BlockSpec((_GT, _E), lambda i: (i, 0)),
            pl.BlockSpec((_GT, _K), lambda i: (i, 0)),
            pl.BlockSpec((_GT, _K), lambda i: (i, 0)),
        ],
        out_shape=[
            jax.ShapeDtypeStruct((_N, _E), jnp.float32),
            jax.ShapeDtypeStruct((_N, _K), jnp.int32),
            jax.ShapeDtypeStruct((_N, _K), jnp.float32),
        ],
        compiler_params=pltpu.CompilerParams(
            dimension_semantics=("arbitrary",)),
    )(x, Wg1, bg1.reshape(1, _H), Wg2, bg2.reshape(1, _E))


def _expert_body(te_ref, xg_ref, we1_ref, be1_ref, we2_ref, be2_ref,
                 wg_ref, yg_ref):
    h = jnp.dot(xg_ref[...], we1_ref[0], preferred_element_type=jnp.float32)
    h = jnp.maximum(h + be1_ref[0], 0.0)
    y = jnp.dot(h, we2_ref[0], preferred_element_type=jnp.float32)
    y = y + be2_ref[0]
    yg_ref[...] = y * wg_ref[...]


def _experts(tile_expert, xg, wg, We1, be1, We2, be2):
    grid_spec = pltpu.PrefetchScalarGridSpec(
        num_scalar_prefetch=1,
        grid=(_T,),
        in_specs=[
            pl.BlockSpec((_TILE, _D), lambda t, te: (t, 0)),
            pl.BlockSpec((1, _D, _H), lambda t, te: (te[t], 0, 0)),
            pl.BlockSpec((1, 1, _H), lambda t, te: (te[t], 0, 0)),
            pl.BlockSpec((1, _H, _D), lambda t, te: (te[t], 0, 0)),
            pl.BlockSpec((1, 1, _D), lambda t, te: (te[t], 0, 0)),
            pl.BlockSpec((_TILE, 1), lambda t, te: (t, 0)),
        ],
        out_specs=pl.BlockSpec((_TILE, _D), lambda t, te: (t, 0)),
    )
    return pl.pallas_call(
        _expert_body,
        grid_spec=grid_spec,
        out_shape=jax.ShapeDtypeStruct((_P, _D), jnp.float32),
        compiler_params=pltpu.CompilerParams(
            dimension_semantics=("arbitrary",)),
    )(tile_expert, xg, We1, be1.reshape(_E, 1, _H), We2,
      be2.reshape(_E, 1, _D), wg)


@functools.lru_cache(maxsize=None)
def _sc_kernels():
    mesh = plsc.VectorSubcoreMesh(core_axis_name="c", subcore_axis_name="s")

    @functools.partial(
        pl.kernel,
        mesh=mesh,
        out_type=jax.ShapeDtypeStruct((_P, _D), jnp.float32),
        scratch_types=[
            pltpu.VMEM((_BPW,), jnp.int32),
            pltpu.VMEM((_GCH, _D), jnp.float32),
            pltpu.SemaphoreType.DMA,
        ],
    )
    def sc_gather(tok_hbm, x_hbm, out_hbm, idx_v, rows_v, sem):
        wid = lax.axis_index("s") * 2 + lax.axis_index("c")
        base = wid * _BPW
        pltpu.sync_copy(tok_hbm.at[pl.ds(base, _BPW)], idx_v)

        def body(i, carry):
            pltpu.async_copy(
                x_hbm.at[idx_v.at[pl.ds(i * _GCH, _GCH)]], rows_v, sem).wait()
            pltpu.sync_copy(rows_v, out_hbm.at[pl.ds(base + i * _GCH, _GCH)])
            return carry

        lax.fori_loop(0, _BPW // _GCH, body, 0)

    @functools.partial(
        pl.kernel,
        mesh=mesh,
        out_type=jax.ShapeDtypeStruct((_N, _D), jnp.float32),
        scratch_types=[
            pltpu.VMEM((_TPW,), jnp.int32),
            pltpu.VMEM((_TPW,), jnp.int32),
            pltpu.VMEM((_CCH, _D), jnp.float32),
            pltpu.VMEM((_CCH, _D), jnp.float32),
            pltpu.SemaphoreType.DMA,
            pltpu.SemaphoreType.DMA,
        ],
    )
    def sc_combine(inv0_hbm, inv1_hbm, yg_hbm, out_hbm,
                   i0_v, i1_v, a_v, b_v, sa, sb):
        wid = lax.axis_index("s") * 2 + lax.axis_index("c")
        base = wid * _TPW
        pltpu.sync_copy(inv0_hbm.at[pl.ds(base, _TPW)], i0_v)
        pltpu.sync_copy(inv1_hbm.at[pl.ds(base, _TPW)], i1_v)

        def chunk(ci, carry):
            ca = pltpu.async_copy(
                yg_hbm.at[i0_v.at[pl.ds(ci * _CCH, _CCH)]], a_v, sa)
            cb = pltpu.async_copy(
                yg_hbm.at[i1_v.at[pl.ds(ci * _CCH, _CCH)]], b_v, sb)
            ca.wait()
            cb.wait()

            def row(r, c2):
                for c in range(_D // 16):
                    sl = pl.ds(c * 16, 16)
                    a_v[r, sl] = a_v[r, sl] + b_v[r, sl]
                return c2

            lax.fori_loop(0, _CCH, row, 0)
            pltpu.sync_copy(a_v, out_hbm.at[pl.ds(base + ci * _CCH, _CCH)])
            return carry

        lax.fori_loop(0, _TPW // _CCH, chunk, 0)

    return sc_gather, sc_combine


def kernel(x, Wg1, bg1, Wg2, bg2, We1, be1, We2, be2):
    probs, idx2, w2 = _gate(x, Wg1, bg1, Wg2, bg2)

    e_flat = idx2.reshape(-1)
    oh = (e_flat[:, None] == jnp.arange(_E, dtype=jnp.int32)[None, :])
    cum = jnp.cumsum(oh.astype(jnp.int32), axis=0)
    rank = jnp.take_along_axis(cum, e_flat[:, None], axis=1)[:, 0] - 1
    counts = cum[-1]
    tiles_e = (counts + _TILE - 1) // _TILE
    cum_tiles = jnp.cumsum(tiles_e)
    pad_base = (cum_tiles - tiles_e) * _TILE
    pos = pad_base[e_flat] + rank
    tok = jnp.arange(_N * _K, dtype=jnp.int32) // _K
    tok_padded = jnp.zeros((_P,), jnp.int32).at[pos].set(tok)
    w_padded = jnp.zeros((_P,), jnp.float32).at[pos].set(w2.reshape(-1))
    tile_expert = jnp.minimum(
        jnp.sum(jnp.arange(_T, dtype=jnp.int32)[:, None] >= cum_tiles[None, :],
                axis=1),
        _E - 1).astype(jnp.int32)

    sc_gather, sc_combine = _sc_kernels()
    xg = sc_gather(tok_padded, x)
    yg = _experts(tile_expert, xg, w_padded.reshape(_P, 1), We1, be1, We2, be2)
    inv = pos.reshape(_N, _K)
    out = sc_combine(inv[:, 0], inv[:, 1], yg)
    return (out, probs)
